# baseline (speedup 1.0000x reference)
import functools

import jax
import jax.numpy as jnp
from jax import lax
from jax.experimental import pallas as pl
from jax.experimental.pallas import tpu as pltpu
from jax.experimental.pallas import tpu_sc as plsc

C = 256
N_ROIS = 1000
N_PAD = 1024
N_BINS = 49
N_TAPS = 4
TOT_TAPS = N_PAD * N_BINS * N_TAPS
TOT_ROWS = N_PAD * N_BINS
D_IN = C * N_BINS

_LVL_H = (192.0, 96.0, 48.0, 24.0)
_LVL_W = (320.0, 160.0, 80.0, 40.0)
_LVL_OFF = (0, 61440, 76800, 80640)
_TABLE_ROWS = 81600
_LVL_SCALE = (0.25, 0.125, 0.0625, 0.03125)


def _index_body(boxes_ref, i00, i01, i10, i11, w00, w01, w10, w11):
    b = boxes_ref[...]
    x1 = b[0:1, :]
    y1 = b[1:2, :]
    x2 = b[2:3, :]
    y2 = b[3:4, :]
    ws = jnp.maximum(x2 - x1, 0.0)
    hs = jnp.maximum(y2 - y1, 0.0)
    area = ws * hs
    lvlf = jnp.floor(4.0 + jnp.log2(jnp.sqrt(area) / 224.0 + 1e-8))
    lvl = jnp.clip(lvlf, 2.0, 5.0).astype(jnp.int32) - 2

    def sel(vals, dtype):
        out = jnp.full(lvl.shape, vals[3], dtype)
        for l in (2, 1, 0):
            out = jnp.where(lvl == l, jnp.asarray(vals[l], dtype), out)
        return out

    scale = sel(_LVL_SCALE, jnp.float32)
    Hf = sel(_LVL_H, jnp.float32)
    Wf = sel(_LVL_W, jnp.float32)
    off = sel(_LVL_OFF, jnp.int32)
    Wi = Wf.astype(jnp.int32)

    x1s = x1 * scale - 0.5
    y1s = y1 * scale - 0.5
    x2s = x2 * scale - 0.5
    y2s = y2 * scale - 0.5
    bw = (x2s - x1s) / 7.0
    bh = (y2s - y1s) / 7.0

    shape = (N_BINS, N_PAD)
    ibf = lax.broadcasted_iota(jnp.int32, shape, 0).astype(jnp.float32)
    iyw = jnp.floor(ibf * (1.0 / 7.0))
    iy = iyw + 0.5
    ix = (ibf - 7.0 * iyw) + 0.5
    yy = y1s + iy * bh
    xx = x1s + ix * bw
    valid = (yy > -1.0) & (yy < Hf) & (xx > -1.0) & (xx < Wf)
    y = jnp.clip(yy, 0.0, Hf - 1.0)
    x = jnp.clip(xx, 0.0, Wf - 1.0)
    y0f = jnp.floor(y)
    x0f = jnp.floor(x)
    y1f = jnp.minimum(y0f + 1.0, Hf - 1.0)
    x1f = jnp.minimum(x0f + 1.0, Wf - 1.0)
    ly = y - y0f
    lx = x - x0f
    vf = valid.astype(jnp.float32)
    w00[...] = (1.0 - ly) * (1.0 - lx) * vf
    w01[...] = (1.0 - ly) * lx * vf
    w10[...] = ly * (1.0 - lx) * vf
    w11[...] = ly * lx * vf
    y0i = y0f.astype(jnp.int32)
    x0i = x0f.astype(jnp.int32)
    y1i = y1f.astype(jnp.int32)
    x1i = x1f.astype(jnp.int32)
    i00[...] = off + y0i * Wi + x0i
    i01[...] = off + y0i * Wi + x1i
    i10[...] = off + y1i * Wi + x0i
    i11[...] = off + y1i * Wi + x1i


def _compute_indices(boxes_pT):
    sh_i = jax.ShapeDtypeStruct((N_BINS, N_PAD), jnp.int32)
    sh_f = jax.ShapeDtypeStruct((N_BINS, N_PAD), jnp.float32)
    return pl.pallas_call(
        _index_body,
        out_shape=(sh_i, sh_i, sh_i, sh_i, sh_f, sh_f, sh_f, sh_f),
    )(boxes_pT)


_NW = 32
_RPW = TOT_TAPS // _NW
_CH = 64
_NCH = _RPW // _CH
_NBUF = 7


def _sc_gather(table, idx):
    mesh = plsc.VectorSubcoreMesh(core_axis_name="c", subcore_axis_name="s")

    @functools.partial(
        pl.kernel,
        out_type=jax.ShapeDtypeStruct((TOT_TAPS, C // 2), jnp.int32),
        mesh=mesh,
        scratch_types=[
            pltpu.VMEM((_RPW,), jnp.int32),
            [pltpu.VMEM((_CH, C // 2), jnp.int32)] * _NBUF,
            [pltpu.SemaphoreType.DMA] * _NBUF,
        ],
    )
    def gk(table_hbm, idx_hbm, out_hbm, idx_v, bufs, sems):
        wid = lax.axis_index("s") * 2 + lax.axis_index("c")
        base = pl.multiple_of(wid * _RPW, 8)
        pltpu.sync_copy(idx_hbm.at[pl.ds(base, _RPW)], idx_v)

        def start(c, j):
            off = pl.multiple_of(c * _CH, 8)
            pltpu.async_copy(
                table_hbm.at[idx_v.at[pl.ds(off, _CH)]], bufs[j], sems[j])

        def finish(c, j):
            off = pl.multiple_of(c * _CH, 8)
            pltpu.make_async_copy(
                table_hbm.at[idx_v.at[pl.ds(off, _CH)]], bufs[j],
                sems[j]).wait()
            dst = pl.multiple_of(base + c * _CH, 8)
            pltpu.sync_copy(bufs[j], out_hbm.at[pl.ds(dst, _CH)])

        for j in range(_NBUF):
            start(j, j)

        def body(g, carry):
            for j in range(_NBUF):
                c = g * _NBUF + j
                finish(c, j)
                start(c + _NBUF, j)
            return carry

        lax.fori_loop(0, _NCH // _NBUF - 1, body, 0)
        for j in range(_NBUF):
            finish(_NCH - _NBUF + j, j)

    return gk(table, idx)


def _tpose_body(f_ref, out_ref):
    xb = f_ref[...].astype(jnp.bfloat16)
    lo = lax.bitcast_convert_type(xb[:C // 2, :], jnp.uint16).astype(
        jnp.uint32)
    hi = lax.bitcast_convert_type(xb[C // 2:, :], jnp.uint16).astype(
        jnp.uint32)
    w = lax.bitcast_convert_type(lo | (hi << 16), jnp.int32)
    out_ref[...] = w.T


def _tpose_pack(fm, blk):
    hw = fm.shape[1]
    return pl.pallas_call(
        _tpose_body,
        grid=(hw // blk,),
        in_specs=[pl.BlockSpec((C, blk), lambda i: (0, i))],
        out_specs=pl.BlockSpec((blk, C // 2), lambda i: (i, 0)),
        out_shape=jax.ShapeDtypeStruct((hw, C // 2), jnp.int32),
    )(fm)


_WBLK = 4096


def _weight_body(g_ref, w_ref, out_ref):
    ylo = jnp.zeros((N_PAD, C // 2), jnp.float32)
    yhi = jnp.zeros((N_PAD, C // 2), jnp.float32)
    for t in range(N_TAPS):
        g = g_ref[t]
        w = w_ref[:, t:t + 1]
        ylo += w * lax.bitcast_convert_type(g << 16, jnp.float32)
        yhi += w * lax.bitcast_convert_type(
            g & jnp.int32(-65536), jnp.float32)
    out_ref[...] = jnp.concatenate(
        [ylo, yhi], axis=1).astype(jnp.bfloat16)


def _weight_reduce(gathered, wts4):
    g3 = gathered.reshape(N_TAPS, TOT_ROWS, C // 2)
    return pl.pallas_call(
        _weight_body,
        grid=(N_BINS,),
        in_specs=[
            pl.BlockSpec((N_TAPS, N_PAD, C // 2), lambda i: (0, i, 0)),
            pl.BlockSpec((N_PAD, N_TAPS), lambda i: (i, 0)),
        ],
        out_specs=pl.BlockSpec((N_PAD, C), lambda i: (i, 0)),
        out_shape=jax.ShapeDtypeStruct((TOT_ROWS, C), jnp.bfloat16),
    )(g3, wts4)


_KB = 1792
_NKB = D_IN // _KB


def _mlp_body(g_ref, w4_ref, w1_ref, b1_ref, w2_ref, b2_ref, wc_ref, bc_ref,
              wd_ref, bd_ref, cls_ref, dlt_ref, acc_ref):
    k = pl.program_id(0)
    ylo = jnp.zeros((N_PAD, C // 2), jnp.float32)
    yhi = jnp.zeros((N_PAD, C // 2), jnp.float32)
    for t in range(N_TAPS):
        g = g_ref[t, 0]
        w = w4_ref[:, t:t + 1]
        ylo += w * lax.bitcast_convert_type(g << 16, jnp.float32)
        yhi += w * lax.bitcast_convert_type(
            g & jnp.int32(-65536), jnp.float32)
    x = jnp.concatenate([ylo, yhi], axis=1).astype(jnp.bfloat16)
    part = jnp.dot(x, w1_ref[0], preferred_element_type=jnp.float32)

    @pl.when(k == 0)
    def _():
        acc_ref[...] = part

    @pl.when(k > 0)
    def _():
        acc_ref[...] = acc_ref[...] + part

    @pl.when(k == N_BINS - 1)
    def _():
        h1 = jnp.maximum(acc_ref[...] + b1_ref[...], 0.0)
        h1b = h1.astype(jnp.bfloat16)
        h2 = jnp.maximum(
            jnp.dot(h1b, w2_ref[...].astype(jnp.bfloat16),
                    preferred_element_type=jnp.float32)
            + b2_ref[...], 0.0)
        h2b = h2.astype(jnp.bfloat16)
        cls_ref[...] = jnp.dot(
            h2b, wc_ref[...].astype(jnp.bfloat16),
            preferred_element_type=jnp.float32) + bc_ref[...]
        dlt_ref[...] = jnp.dot(
            h2b, wd_ref[...].astype(jnp.bfloat16),
            preferred_element_type=jnp.float32) + bd_ref[...]


def _mlp(g4, wts4, W1vb, b1, W2, b2, Wcp, bcp, Wdp, bdp):
    return pl.pallas_call(
        _mlp_body,
        grid=(N_BINS,),
        in_specs=[
            pl.BlockSpec((N_TAPS, 1, N_PAD, C // 2), lambda k: (0, k, 0, 0)),
            pl.BlockSpec((N_PAD, N_TAPS), lambda k: (k, 0)),
            pl.BlockSpec((1, C, 1024), lambda k: (k, 0, 0)),
            pl.BlockSpec((1, 1024), lambda k: (0, 0)),
            pl.BlockSpec((1024, 1024), lambda k: (0, 0)),
            pl.BlockSpec((1, 1024), lambda k: (0, 0)),
            pl.BlockSpec((1024, 128), lambda k: (0, 0)),
            pl.BlockSpec((1, 128), lambda k: (0, 0)),
            pl.BlockSpec((1024, 384), lambda k: (0, 0)),
            pl.BlockSpec((1, 384), lambda k: (0, 0)),
        ],
        out_specs=(
            pl.BlockSpec((N_PAD, 128), lambda k: (0, 0)),
            pl.BlockSpec((N_PAD, 384), lambda k: (0, 0)),
        ),
        out_shape=(
            jax.ShapeDtypeStruct((N_PAD, 128), jnp.float32),
            jax.ShapeDtypeStruct((N_PAD, 384), jnp.float32),
        ),
        scratch_shapes=[pltpu.VMEM((N_PAD, 1024), jnp.float32)],
        compiler_params=pltpu.CompilerParams(
            dimension_semantics=("arbitrary",)),
    )(g4, wts4, W1vb, b1, W2, b2, Wcp, bcp, Wdp, bdp)


def kernel(fm0, fm1, fm2, fm3, fm4, rcnn_rois, W1, b1, W2, b2, Wc, bc, Wd, bd):
    table = jnp.concatenate([
        _tpose_pack(fm4[0].reshape(C, -1), 1920),
        _tpose_pack(fm3[0].reshape(C, -1), 1920),
        _tpose_pack(fm2[0].reshape(C, -1), 1920),
        _tpose_pack(fm1[0].reshape(C, -1), 960),
    ], axis=0)

    boxes_pT = jnp.concatenate(
        [rcnn_rois[:, 1:5], jnp.zeros((N_PAD - N_ROIS, 4), jnp.float32)],
        axis=0).T

    i00, i01, i10, i11, w00, w01, w10, w11 = _compute_indices(boxes_pT)
    idx = jnp.concatenate(
        [i00.reshape(-1), i01.reshape(-1), i10.reshape(-1),
         i11.reshape(-1)])
    wts4 = jnp.concatenate(
        [w00.reshape(1, -1), w01.reshape(1, -1), w10.reshape(1, -1),
         w11.reshape(1, -1)], axis=0).T

    gathered = _sc_gather(table, idx)
    g4 = gathered.reshape(N_TAPS, N_BINS, N_PAD, C // 2)

    W1vb = W1.reshape(C, N_BINS, 1024).transpose(1, 0, 2).astype(
        jnp.bfloat16)
    Wcp = jnp.pad(Wc, ((0, 0), (0, 128 - 81)))
    bcp = jnp.pad(bc, (0, 128 - 81)).reshape(1, 128)
    Wdp = jnp.pad(Wd, ((0, 0), (0, 384 - 324)))
    bdp = jnp.pad(bd, (0, 384 - 324)).reshape(1, 384)

    cls, dlt = _mlp(g4, wts4, W1vb, b1.reshape(1, 1024), W2,
                    b2.reshape(1, 1024), Wcp, bcp, Wdp, bdp)
    return cls[:N_ROIS, :81], dlt[:N_ROIS, :324]


# speedup vs baseline: 1.0160x; 1.0160x over previous
import functools

import jax
import jax.numpy as jnp
from jax import lax
from jax.experimental import pallas as pl
from jax.experimental.pallas import tpu as pltpu
from jax.experimental.pallas import tpu_sc as plsc

C = 256
N_ROIS = 1000
N_PAD = 1024
N_BINS = 49
N_TAPS = 4
TOT_TAPS = N_PAD * N_BINS * N_TAPS
TOT_ROWS = N_PAD * N_BINS
D_IN = C * N_BINS

_LVL_H = (192.0, 96.0, 48.0, 24.0)
_LVL_W = (320.0, 160.0, 80.0, 40.0)
_LVL_OFF = (0, 61440, 76800, 80640)
_TABLE_ROWS = 81600
_LVL_SCALE = (0.25, 0.125, 0.0625, 0.03125)


def _index_body(boxes_ref, i00, i01, i10, i11, w00, w01, w10, w11):
    b = boxes_ref[...]
    x1 = b[0:1, :]
    y1 = b[1:2, :]
    x2 = b[2:3, :]
    y2 = b[3:4, :]
    ws = jnp.maximum(x2 - x1, 0.0)
    hs = jnp.maximum(y2 - y1, 0.0)
    area = ws * hs
    lvlf = jnp.floor(4.0 + jnp.log2(jnp.sqrt(area) / 224.0 + 1e-8))
    lvl = jnp.clip(lvlf, 2.0, 5.0).astype(jnp.int32) - 2

    def sel(vals, dtype):
        out = jnp.full(lvl.shape, vals[3], dtype)
        for l in (2, 1, 0):
            out = jnp.where(lvl == l, jnp.asarray(vals[l], dtype), out)
        return out

    scale = sel(_LVL_SCALE, jnp.float32)
    Hf = sel(_LVL_H, jnp.float32)
    Wf = sel(_LVL_W, jnp.float32)
    off = sel(_LVL_OFF, jnp.int32)
    Wi = Wf.astype(jnp.int32)

    x1s = x1 * scale - 0.5
    y1s = y1 * scale - 0.5
    x2s = x2 * scale - 0.5
    y2s = y2 * scale - 0.5
    bw = (x2s - x1s) / 7.0
    bh = (y2s - y1s) / 7.0

    shape = (N_BINS, N_PAD)
    ibf = lax.broadcasted_iota(jnp.int32, shape, 0).astype(jnp.float32)
    iyw = jnp.floor(ibf * (1.0 / 7.0))
    iy = iyw + 0.5
    ix = (ibf - 7.0 * iyw) + 0.5
    yy = y1s + iy * bh
    xx = x1s + ix * bw
    valid = (yy > -1.0) & (yy < Hf) & (xx > -1.0) & (xx < Wf)
    y = jnp.clip(yy, 0.0, Hf - 1.0)
    x = jnp.clip(xx, 0.0, Wf - 1.0)
    y0f = jnp.floor(y)
    x0f = jnp.floor(x)
    y1f = jnp.minimum(y0f + 1.0, Hf - 1.0)
    x1f = jnp.minimum(x0f + 1.0, Wf - 1.0)
    ly = y - y0f
    lx = x - x0f
    vf = valid.astype(jnp.float32)
    w00[...] = (1.0 - ly) * (1.0 - lx) * vf
    w01[...] = (1.0 - ly) * lx * vf
    w10[...] = ly * (1.0 - lx) * vf
    w11[...] = ly * lx * vf
    y0i = y0f.astype(jnp.int32)
    x0i = x0f.astype(jnp.int32)
    y1i = y1f.astype(jnp.int32)
    x1i = x1f.astype(jnp.int32)
    i00[...] = off + y0i * Wi + x0i
    i01[...] = off + y0i * Wi + x1i
    i10[...] = off + y1i * Wi + x0i
    i11[...] = off + y1i * Wi + x1i


def _compute_indices(boxes_pT):
    sh_i = jax.ShapeDtypeStruct((N_BINS, N_PAD), jnp.int32)
    sh_f = jax.ShapeDtypeStruct((N_BINS, N_PAD), jnp.float32)
    return pl.pallas_call(
        _index_body,
        out_shape=(sh_i, sh_i, sh_i, sh_i, sh_f, sh_f, sh_f, sh_f),
    )(boxes_pT)


_NW = 32
_RPW = TOT_TAPS // _NW
_CH = 64


def _sc_gather(table, idx, n_taps, nbuf):
    rpw = n_taps // _NW
    nch = rpw // _CH
    mesh = plsc.VectorSubcoreMesh(core_axis_name="c", subcore_axis_name="s")

    @functools.partial(
        pl.kernel,
        out_type=jax.ShapeDtypeStruct((n_taps, C // 2), jnp.int32),
        mesh=mesh,
        scratch_types=[
            pltpu.VMEM((rpw,), jnp.int32),
            [pltpu.VMEM((_CH, C // 2), jnp.int32)] * nbuf,
            [pltpu.SemaphoreType.DMA] * nbuf,
        ],
    )
    def gk(table_hbm, idx_hbm, out_hbm, idx_v, bufs, sems):
        wid = lax.axis_index("s") * 2 + lax.axis_index("c")
        base = pl.multiple_of(wid * rpw, 8)
        pltpu.sync_copy(idx_hbm.at[pl.ds(base, rpw)], idx_v)

        def start(c, j):
            off = pl.multiple_of(c * _CH, 8)
            pltpu.async_copy(
                table_hbm.at[idx_v.at[pl.ds(off, _CH)]], bufs[j], sems[j])

        def finish(c, j):
            off = pl.multiple_of(c * _CH, 8)
            pltpu.make_async_copy(
                table_hbm.at[idx_v.at[pl.ds(off, _CH)]], bufs[j],
                sems[j]).wait()
            dst = pl.multiple_of(base + c * _CH, 8)
            pltpu.sync_copy(bufs[j], out_hbm.at[pl.ds(dst, _CH)])

        for j in range(nbuf):
            start(j, j)

        def body(g, carry):
            for j in range(nbuf):
                c = g * nbuf + j
                finish(c, j)
                start(c + nbuf, j)
            return carry

        lax.fori_loop(0, nch // nbuf - 1, body, 0)
        for j in range(nbuf):
            finish(nch - nbuf + j, j)

    return gk(table, idx)


def _tpose_body(f_ref, out_ref):
    xb = f_ref[...].astype(jnp.bfloat16)
    lo = lax.bitcast_convert_type(xb[:C // 2, :], jnp.uint16).astype(
        jnp.uint32)
    hi = lax.bitcast_convert_type(xb[C // 2:, :], jnp.uint16).astype(
        jnp.uint32)
    w = lax.bitcast_convert_type(lo | (hi << 16), jnp.int32)
    out_ref[...] = w.T


def _tpose_pack(fm, blk):
    hw = fm.shape[1]
    return pl.pallas_call(
        _tpose_body,
        grid=(hw // blk,),
        in_specs=[pl.BlockSpec((C, blk), lambda i: (0, i))],
        out_specs=pl.BlockSpec((blk, C // 2), lambda i: (i, 0)),
        out_shape=jax.ShapeDtypeStruct((hw, C // 2), jnp.int32),
    )(fm)


_WBLK = 4096


def _weight_body(g_ref, w_ref, out_ref):
    ylo = jnp.zeros((N_PAD, C // 2), jnp.float32)
    yhi = jnp.zeros((N_PAD, C // 2), jnp.float32)
    for t in range(N_TAPS):
        g = g_ref[t]
        w = w_ref[:, t:t + 1]
        ylo += w * lax.bitcast_convert_type(g << 16, jnp.float32)
        yhi += w * lax.bitcast_convert_type(
            g & jnp.int32(-65536), jnp.float32)
    out_ref[...] = jnp.concatenate(
        [ylo, yhi], axis=1).astype(jnp.bfloat16)


def _weight_reduce(gathered, wts4):
    g3 = gathered.reshape(N_TAPS, TOT_ROWS, C // 2)
    return pl.pallas_call(
        _weight_body,
        grid=(N_BINS,),
        in_specs=[
            pl.BlockSpec((N_TAPS, N_PAD, C // 2), lambda i: (0, i, 0)),
            pl.BlockSpec((N_PAD, N_TAPS), lambda i: (i, 0)),
        ],
        out_specs=pl.BlockSpec((N_PAD, C), lambda i: (i, 0)),
        out_shape=jax.ShapeDtypeStruct((TOT_ROWS, C), jnp.bfloat16),
    )(g3, wts4)


_KB = 1792
_NKB = D_IN // _KB


def _fc1_body(nb, g_ref, w4_ref, w1_ref, out_ref, acc_ref):
    k = pl.program_id(0)
    ylo = jnp.zeros((N_PAD, C // 2), jnp.float32)
    yhi = jnp.zeros((N_PAD, C // 2), jnp.float32)
    for t in range(N_TAPS):
        g = g_ref[t, 0]
        w = w4_ref[:, t:t + 1]
        ylo += w * lax.bitcast_convert_type(g << 16, jnp.float32)
        yhi += w * lax.bitcast_convert_type(
            g & jnp.int32(-65536), jnp.float32)
    x = jnp.concatenate([ylo, yhi], axis=1).astype(jnp.bfloat16)
    part = jnp.dot(x, w1_ref[0], preferred_element_type=jnp.float32)

    @pl.when(k == 0)
    def _():
        acc_ref[...] = part

    @pl.when(k > 0)
    def _():
        acc_ref[...] = acc_ref[...] + part

    @pl.when(k == nb - 1)
    def _():
        out_ref[...] = acc_ref[...]


def _fc1(g4, wts4, W1vb, nb):
    return pl.pallas_call(
        functools.partial(_fc1_body, nb),
        grid=(nb,),
        in_specs=[
            pl.BlockSpec((N_TAPS, 1, N_PAD, C // 2), lambda k: (0, k, 0, 0)),
            pl.BlockSpec((N_PAD, N_TAPS), lambda k: (k, 0)),
            pl.BlockSpec((1, C, 1024), lambda k: (k, 0, 0)),
        ],
        out_specs=pl.BlockSpec((N_PAD, 1024), lambda k: (0, 0)),
        out_shape=jax.ShapeDtypeStruct((N_PAD, 1024), jnp.float32),
        scratch_shapes=[pltpu.VMEM((N_PAD, 1024), jnp.float32)],
        compiler_params=pltpu.CompilerParams(
            dimension_semantics=("arbitrary",)),
    )(g4, wts4, W1vb)


def _head_body(a_ref, b_ref, b1_ref, w2_ref, b2_ref, wc_ref, bc_ref,
               wd_ref, bd_ref, cls_ref, dlt_ref):
    h1 = jnp.maximum(a_ref[...] + b_ref[...] + b1_ref[...], 0.0)
    h1b = h1.astype(jnp.bfloat16)
    h2 = jnp.maximum(
        jnp.dot(h1b, w2_ref[...].astype(jnp.bfloat16),
                preferred_element_type=jnp.float32)
        + b2_ref[...], 0.0)
    h2b = h2.astype(jnp.bfloat16)
    cls_ref[...] = jnp.dot(
        h2b, wc_ref[...].astype(jnp.bfloat16),
        preferred_element_type=jnp.float32) + bc_ref[...]
    dlt_ref[...] = jnp.dot(
        h2b, wd_ref[...].astype(jnp.bfloat16),
        preferred_element_type=jnp.float32) + bd_ref[...]


def _head(acc_a, acc_b, b1, W2, b2, Wcp, bcp, Wdp, bdp):
    return pl.pallas_call(
        _head_body,
        out_shape=(
            jax.ShapeDtypeStruct((N_PAD, 128), jnp.float32),
            jax.ShapeDtypeStruct((N_PAD, 384), jnp.float32),
        ),
    )(acc_a, acc_b, b1, W2, b2, Wcp, bcp, Wdp, bdp)


def kernel(fm0, fm1, fm2, fm3, fm4, rcnn_rois, W1, b1, W2, b2, Wc, bc, Wd, bd):
    table = jnp.concatenate([
        _tpose_pack(fm4[0].reshape(C, -1), 1920),
        _tpose_pack(fm3[0].reshape(C, -1), 1920),
        _tpose_pack(fm2[0].reshape(C, -1), 1920),
        _tpose_pack(fm1[0].reshape(C, -1), 960),
    ], axis=0)

    boxes_pT = jnp.concatenate(
        [rcnn_rois[:, 1:5], jnp.zeros((N_PAD - N_ROIS, 4), jnp.float32)],
        axis=0).T

    i00, i01, i10, i11, w00, w01, w10, w11 = _compute_indices(boxes_pT)
    nba, nbb = 24, N_BINS - 24
    idx_a = jnp.concatenate(
        [i00[:nba].reshape(-1), i01[:nba].reshape(-1),
         i10[:nba].reshape(-1), i11[:nba].reshape(-1)])
    idx_b = jnp.concatenate(
        [i00[nba:].reshape(-1), i01[nba:].reshape(-1),
         i10[nba:].reshape(-1), i11[nba:].reshape(-1)])
    wts4_a = jnp.concatenate(
        [w00[:nba].reshape(1, -1), w01[:nba].reshape(1, -1),
         w10[:nba].reshape(1, -1), w11[:nba].reshape(1, -1)], axis=0).T
    wts4_b = jnp.concatenate(
        [w00[nba:].reshape(1, -1), w01[nba:].reshape(1, -1),
         w10[nba:].reshape(1, -1), w11[nba:].reshape(1, -1)], axis=0).T

    gathered_a = _sc_gather(table, idx_a, N_TAPS * nba * N_PAD, 4)
    gathered_b = _sc_gather(table, idx_b, N_TAPS * nbb * N_PAD, 5)
    g4_a = gathered_a.reshape(N_TAPS, nba, N_PAD, C // 2)
    g4_b = gathered_b.reshape(N_TAPS, nbb, N_PAD, C // 2)

    W1vb = W1.reshape(C, N_BINS, 1024).transpose(1, 0, 2).astype(
        jnp.bfloat16)
    Wcp = jnp.pad(Wc, ((0, 0), (0, 128 - 81)))
    bcp = jnp.pad(bc, (0, 128 - 81)).reshape(1, 128)
    Wdp = jnp.pad(Wd, ((0, 0), (0, 384 - 324)))
    bdp = jnp.pad(bd, (0, 384 - 324)).reshape(1, 384)

    acc_a = _fc1(g4_a, wts4_a, W1vb[:nba], nba)
    acc_b = _fc1(g4_b, wts4_b, W1vb[nba:], nbb)
    cls, dlt = _head(acc_a, acc_b, b1.reshape(1, 1024), W2,
                     b2.reshape(1, 1024), Wcp, bcp, Wdp, bdp)
    return cls[:N_ROIS, :81], dlt[:N_ROIS, :324]


# speedup vs baseline: 1.0397x; 1.0233x over previous
import functools

import jax
import jax.numpy as jnp
from jax import lax
from jax.experimental import pallas as pl
from jax.experimental.pallas import tpu as pltpu
from jax.experimental.pallas import tpu_sc as plsc

C = 256
N_ROIS = 1000
N_PAD = 1024
N_BINS = 49
N_TAPS = 4
TOT_TAPS = N_PAD * N_BINS * N_TAPS
TOT_ROWS = N_PAD * N_BINS
D_IN = C * N_BINS

_LVL_H = (192.0, 96.0, 48.0, 24.0)
_LVL_W = (320.0, 160.0, 80.0, 40.0)
_LVL_OFF = (0, 61440, 76800, 80640)
_TABLE_ROWS = 81600
_LVL_SCALE = (0.25, 0.125, 0.0625, 0.03125)


def _index_body(boxes_ref, i00, i01, i10, i11, w00, w01, w10, w11):
    b = boxes_ref[...]
    x1 = b[0:1, :]
    y1 = b[1:2, :]
    x2 = b[2:3, :]
    y2 = b[3:4, :]
    ws = jnp.maximum(x2 - x1, 0.0)
    hs = jnp.maximum(y2 - y1, 0.0)
    area = ws * hs
    lvlf = jnp.floor(4.0 + jnp.log2(jnp.sqrt(area) / 224.0 + 1e-8))
    lvl = jnp.clip(lvlf, 2.0, 5.0).astype(jnp.int32) - 2

    def sel(vals, dtype):
        out = jnp.full(lvl.shape, vals[3], dtype)
        for l in (2, 1, 0):
            out = jnp.where(lvl == l, jnp.asarray(vals[l], dtype), out)
        return out

    scale = sel(_LVL_SCALE, jnp.float32)
    Hf = sel(_LVL_H, jnp.float32)
    Wf = sel(_LVL_W, jnp.float32)
    off = sel(_LVL_OFF, jnp.int32)
    Wi = Wf.astype(jnp.int32)

    x1s = x1 * scale - 0.5
    y1s = y1 * scale - 0.5
    x2s = x2 * scale - 0.5
    y2s = y2 * scale - 0.5
    bw = (x2s - x1s) / 7.0
    bh = (y2s - y1s) / 7.0

    shape = (N_BINS, N_PAD)
    ibf = lax.broadcasted_iota(jnp.int32, shape, 0).astype(jnp.float32)
    iyw = jnp.floor(ibf * (1.0 / 7.0))
    iy = iyw + 0.5
    ix = (ibf - 7.0 * iyw) + 0.5
    yy = y1s + iy * bh
    xx = x1s + ix * bw
    valid = (yy > -1.0) & (yy < Hf) & (xx > -1.0) & (xx < Wf)
    y = jnp.clip(yy, 0.0, Hf - 1.0)
    x = jnp.clip(xx, 0.0, Wf - 1.0)
    y0f = jnp.floor(y)
    x0f = jnp.floor(x)
    y1f = jnp.minimum(y0f + 1.0, Hf - 1.0)
    x1f = jnp.minimum(x0f + 1.0, Wf - 1.0)
    ly = y - y0f
    lx = x - x0f
    vf = valid.astype(jnp.float32)
    w00[...] = (1.0 - ly) * (1.0 - lx) * vf
    w01[...] = (1.0 - ly) * lx * vf
    w10[...] = ly * (1.0 - lx) * vf
    w11[...] = ly * lx * vf
    y0i = y0f.astype(jnp.int32)
    x0i = x0f.astype(jnp.int32)
    y1i = y1f.astype(jnp.int32)
    x1i = x1f.astype(jnp.int32)
    i00[...] = off + y0i * Wi + x0i
    i01[...] = off + y0i * Wi + x1i
    i10[...] = off + y1i * Wi + x0i
    i11[...] = off + y1i * Wi + x1i


def _compute_indices(boxes_pT):
    sh_i = jax.ShapeDtypeStruct((N_BINS, N_PAD), jnp.int32)
    sh_f = jax.ShapeDtypeStruct((N_BINS, N_PAD), jnp.float32)
    return pl.pallas_call(
        _index_body,
        out_shape=(sh_i, sh_i, sh_i, sh_i, sh_f, sh_f, sh_f, sh_f),
    )(boxes_pT)


_NW = 32
_RPW = TOT_TAPS // _NW
_CH = 64


def _sc_gather(table, idx, n_taps, nbuf):
    rpw = n_taps // _NW
    nch = rpw // _CH
    mesh = plsc.VectorSubcoreMesh(core_axis_name="c", subcore_axis_name="s")

    @functools.partial(
        pl.kernel,
        out_type=jax.ShapeDtypeStruct((n_taps, C // 2), jnp.int32),
        mesh=mesh,
        scratch_types=[
            pltpu.VMEM((rpw,), jnp.int32),
            [pltpu.VMEM((_CH, C // 2), jnp.int32)] * nbuf,
            [pltpu.SemaphoreType.DMA] * nbuf,
        ],
    )
    def gk(table_hbm, idx_hbm, out_hbm, idx_v, bufs, sems):
        wid = lax.axis_index("s") * 2 + lax.axis_index("c")
        base = pl.multiple_of(wid * rpw, 8)
        pltpu.sync_copy(idx_hbm.at[pl.ds(base, rpw)], idx_v)

        def start(c, j):
            off = pl.multiple_of(c * _CH, 8)
            pltpu.async_copy(
                table_hbm.at[idx_v.at[pl.ds(off, _CH)]], bufs[j], sems[j])

        def finish(c, j):
            off = pl.multiple_of(c * _CH, 8)
            pltpu.make_async_copy(
                table_hbm.at[idx_v.at[pl.ds(off, _CH)]], bufs[j],
                sems[j]).wait()
            dst = pl.multiple_of(base + c * _CH, 8)
            pltpu.sync_copy(bufs[j], out_hbm.at[pl.ds(dst, _CH)])

        for j in range(nbuf):
            start(j, j)

        def body(g, carry):
            for j in range(nbuf):
                c = g * nbuf + j
                finish(c, j)
                start(c + nbuf, j)
            return carry

        lax.fori_loop(0, nch // nbuf - 1, body, 0)
        for j in range(nbuf):
            finish(nch - nbuf + j, j)

    return gk(table, idx)


def _tpose_body(f_ref, out_ref):
    xb = f_ref[...].astype(jnp.bfloat16)
    lo = lax.bitcast_convert_type(xb[:C // 2, :], jnp.uint16).astype(
        jnp.uint32)
    hi = lax.bitcast_convert_type(xb[C // 2:, :], jnp.uint16).astype(
        jnp.uint32)
    w = lax.bitcast_convert_type(lo | (hi << 16), jnp.int32)
    out_ref[...] = w.T


def _tpose_pack(fm, blk):
    hw = fm.shape[1]
    return pl.pallas_call(
        _tpose_body,
        grid=(hw // blk,),
        in_specs=[pl.BlockSpec((C, blk), lambda i: (0, i))],
        out_specs=pl.BlockSpec((blk, C // 2), lambda i: (i, 0)),
        out_shape=jax.ShapeDtypeStruct((hw, C // 2), jnp.int32),
    )(fm)


_WBLK = 4096


def _weight_body(g_ref, w_ref, out_ref):
    ylo = jnp.zeros((N_PAD, C // 2), jnp.float32)
    yhi = jnp.zeros((N_PAD, C // 2), jnp.float32)
    for t in range(N_TAPS):
        g = g_ref[t]
        w = w_ref[:, t:t + 1]
        ylo += w * lax.bitcast_convert_type(g << 16, jnp.float32)
        yhi += w * lax.bitcast_convert_type(
            g & jnp.int32(-65536), jnp.float32)
    out_ref[...] = jnp.concatenate(
        [ylo, yhi], axis=1).astype(jnp.bfloat16)


def _weight_reduce(gathered, wts4):
    g3 = gathered.reshape(N_TAPS, TOT_ROWS, C // 2)
    return pl.pallas_call(
        _weight_body,
        grid=(N_BINS,),
        in_specs=[
            pl.BlockSpec((N_TAPS, N_PAD, C // 2), lambda i: (0, i, 0)),
            pl.BlockSpec((N_PAD, N_TAPS), lambda i: (i, 0)),
        ],
        out_specs=pl.BlockSpec((N_PAD, C), lambda i: (i, 0)),
        out_shape=jax.ShapeDtypeStruct((TOT_ROWS, C), jnp.bfloat16),
    )(g3, wts4)


_KB = 1792
_NKB = D_IN // _KB


def _fc1_body(nb, g_ref, w4_ref, w1_ref, out_ref, acc_ref):
    k = pl.program_id(0)
    w4 = w4_ref[:, 0, 0, :].T
    ylo = jnp.zeros((N_PAD, C // 2), jnp.float32)
    yhi = jnp.zeros((N_PAD, C // 2), jnp.float32)
    for t in range(N_TAPS):
        g = g_ref[t, 0]
        w = w4[:, t:t + 1]
        ylo += w * lax.bitcast_convert_type(g << 16, jnp.float32)
        yhi += w * lax.bitcast_convert_type(
            g & jnp.int32(-65536), jnp.float32)
    x = jnp.concatenate([ylo, yhi], axis=1).astype(jnp.bfloat16)
    w1b = w1_ref[:, 0].reshape(C, 1024).astype(jnp.bfloat16)
    part = jnp.dot(x, w1b, preferred_element_type=jnp.float32)

    @pl.when(k == 0)
    def _():
        acc_ref[...] = part

    @pl.when(k > 0)
    def _():
        acc_ref[...] = acc_ref[...] + part

    @pl.when(k == nb - 1)
    def _():
        out_ref[...] = acc_ref[...]


def _fc1(g4, wts_cat, W14, nb, off):
    return pl.pallas_call(
        functools.partial(_fc1_body, nb),
        grid=(nb,),
        in_specs=[
            pl.BlockSpec((N_TAPS, 1, N_PAD, C // 2), lambda k: (0, k, 0, 0)),
            pl.BlockSpec((N_TAPS, 1, 1, N_PAD),
                         lambda k: (0, k + off, 0, 0)),
            pl.BlockSpec((C, 1, 8, 128), lambda k: (0, k + off, 0, 0)),
        ],
        out_specs=pl.BlockSpec((N_PAD, 1024), lambda k: (0, 0)),
        out_shape=jax.ShapeDtypeStruct((N_PAD, 1024), jnp.float32),
        scratch_shapes=[pltpu.VMEM((N_PAD, 1024), jnp.float32)],
        compiler_params=pltpu.CompilerParams(
            dimension_semantics=("arbitrary",)),
    )(g4, wts_cat, W14)


def _head_body(a_ref, b_ref, b1_ref, w2_ref, b2_ref, wc_ref, bc_ref,
               wd_ref, bd_ref, cls_ref, dlt_ref):
    h1 = jnp.maximum(a_ref[...] + b_ref[...] + b1_ref[...], 0.0)
    h1b = h1.astype(jnp.bfloat16)
    h2 = jnp.maximum(
        jnp.dot(h1b, w2_ref[...].astype(jnp.bfloat16),
                preferred_element_type=jnp.float32)
        + b2_ref[...], 0.0)
    h2b = h2.astype(jnp.bfloat16)
    cls_ref[...] = jnp.dot(
        h2b, wc_ref[...].astype(jnp.bfloat16),
        preferred_element_type=jnp.float32) + bc_ref[...]
    dlt_ref[...] = jnp.dot(
        h2b, wd_ref[...].astype(jnp.bfloat16),
        preferred_element_type=jnp.float32) + bd_ref[...]


def _head(acc_a, acc_b, b1, W2, b2, Wcp, bcp, Wdp, bdp):
    return pl.pallas_call(
        _head_body,
        out_shape=(
            jax.ShapeDtypeStruct((N_PAD, 128), jnp.float32),
            jax.ShapeDtypeStruct((N_PAD, 384), jnp.float32),
        ),
    )(acc_a, acc_b, b1, W2, b2, Wcp, bcp, Wdp, bdp)


def kernel(fm0, fm1, fm2, fm3, fm4, rcnn_rois, W1, b1, W2, b2, Wc, bc, Wd, bd):
    table = jnp.concatenate([
        _tpose_pack(fm4[0].reshape(C, -1), 1920),
        _tpose_pack(fm3[0].reshape(C, -1), 1920),
        _tpose_pack(fm2[0].reshape(C, -1), 1920),
        _tpose_pack(fm1[0].reshape(C, -1), 960),
    ], axis=0)

    boxes_pT = jnp.concatenate(
        [rcnn_rois[:, 1:5], jnp.zeros((N_PAD - N_ROIS, 4), jnp.float32)],
        axis=0).T

    i00, i01, i10, i11, w00, w01, w10, w11 = _compute_indices(boxes_pT)
    nba, nbb = 24, N_BINS - 24
    idx_a = jnp.concatenate(
        [i00[:nba].reshape(-1), i01[:nba].reshape(-1),
         i10[:nba].reshape(-1), i11[:nba].reshape(-1)])
    idx_b = jnp.concatenate(
        [i00[nba:].reshape(-1), i01[nba:].reshape(-1),
         i10[nba:].reshape(-1), i11[nba:].reshape(-1)])
    wts_cat = jnp.stack([w00, w01, w10, w11], axis=0).reshape(
        N_TAPS, N_BINS, 1, N_PAD)

    gathered_a = _sc_gather(table, idx_a, N_TAPS * nba * N_PAD, 4)
    gathered_b = _sc_gather(table, idx_b, N_TAPS * nbb * N_PAD, 5)
    g4_a = gathered_a.reshape(N_TAPS, nba, N_PAD, C // 2)
    g4_b = gathered_b.reshape(N_TAPS, nbb, N_PAD, C // 2)

    W14 = W1.reshape(C, N_BINS, 8, 128)
    Wcp = jnp.pad(Wc, ((0, 0), (0, 128 - 81)))
    bcp = jnp.pad(bc, (0, 128 - 81)).reshape(1, 128)
    Wdp = jnp.pad(Wd, ((0, 0), (0, 384 - 324)))
    bdp = jnp.pad(bd, (0, 384 - 324)).reshape(1, 384)

    acc_a = _fc1(g4_a, wts_cat, W14, nba, 0)
    acc_b = _fc1(g4_b, wts_cat, W14, nbb, nba)
    cls, dlt = _head(acc_a, acc_b, b1.reshape(1, 1024), W2,
                     b2.reshape(1, 1024), Wcp, bcp, Wdp, bdp)
    return cls[:N_ROIS, :81], dlt[:N_ROIS, :324]
